# Initial kernel scaffold; baseline (speedup 1.0000x reference)
import jax
import jax.numpy as jnp
from jax.experimental import pallas as pl


def kernel(input_words, in_table):
    raise NotImplementedError("write your pallas kernel here")


# speedup vs baseline: 1.0949x; 1.0949x over previous
import functools

import jax
import jax.numpy as jnp
from jax import lax
from jax.experimental import pallas as pl
from jax.experimental.pallas import tpu as pltpu
from jax.experimental.pallas import tpu_sc as plsc

DIM = 32
NC = 2
NS = 16
NW = NC * NS

IDXW = 128
K = 8
CHUNK = K * IDXW


def _make_gather(n_rows):
    per_w = n_rows // NW
    n_chunks = per_w // CHUNK
    rows_per_w = per_w // IDXW

    mesh = plsc.VectorSubcoreMesh(core_axis_name="c", subcore_axis_name="s")

    @functools.partial(
        pl.kernel,
        mesh=mesh,
        out_type=jax.ShapeDtypeStruct((n_rows, DIM), jnp.float32),
        scratch_types=[
            pltpu.VMEM((K, IDXW), jnp.int32),
            pltpu.VMEM((CHUNK, DIM), jnp.float32),
            pltpu.SemaphoreType.DMA,
        ],
        compiler_params=pltpu.CompilerParams(use_tc_tiling_on_sc=False),
    )
    def gather_kernel(idx_hbm, table_hbm, out_hbm, idx_v, rows_v, sem):
        wid = lax.axis_index("s") * NC + lax.axis_index("c")
        row_base = wid * rows_per_w

        def body(g, _):
            r0 = row_base + g * K
            pltpu.sync_copy(idx_hbm.at[pl.ds(r0, K)], idx_v)
            copies = [
                pltpu.async_copy(
                    table_hbm.at[idx_v.at[k]],
                    rows_v.at[pl.ds(k * IDXW, IDXW)],
                    sem,
                )
                for k in range(K)
            ]
            for c in copies:
                c.wait()
            pltpu.sync_copy(rows_v, out_hbm.at[pl.ds(r0 * IDXW, CHUNK)])
            return 0

        lax.fori_loop(0, n_chunks, body, 0)

    return gather_kernel


def kernel(input_words, in_table):
    b, l = input_words.shape
    n = b * l
    idx2d = input_words.reshape(n // IDXW, IDXW).astype(jnp.int32)
    out = _make_gather(n)(idx2d, in_table)
    return out.reshape(b, l, DIM)


# speedup vs baseline: 1.1128x; 1.0163x over previous
import functools

import jax
import jax.numpy as jnp
from jax import lax
from jax.experimental import pallas as pl
from jax.experimental.pallas import tpu as pltpu
from jax.experimental.pallas import tpu_sc as plsc

DIM = 32
NC = 2
NS = 16
NW = NC * NS

IDXW = 128
K = 10
CHUNK = K * IDXW
NBUF = 2


def _make_gather(n_rows):
    per_w = n_rows // NW
    n_chunks = per_w // CHUNK
    rows_per_w = per_w // IDXW

    mesh = plsc.VectorSubcoreMesh(core_axis_name="c", subcore_axis_name="s")

    @functools.partial(
        pl.kernel,
        mesh=mesh,
        out_type=jax.ShapeDtypeStruct((n_rows, DIM), jnp.float32),
        scratch_types=[
            pltpu.VMEM((rows_per_w, IDXW), jnp.int32),
            pltpu.VMEM((NBUF, CHUNK, DIM), jnp.float32),
            pltpu.SemaphoreType.DMA((NBUF,)),
            pltpu.SemaphoreType.DMA((NBUF,)),
        ],
        compiler_params=pltpu.CompilerParams(use_tc_tiling_on_sc=False),
    )
    def gather_kernel(idx_hbm, table_hbm, out_hbm, idx_v, rows_v, gsem, osem):
        wid = lax.axis_index("s") * NC + lax.axis_index("c")
        row_base = wid * rows_per_w

        pltpu.sync_copy(idx_hbm.at[pl.ds(row_base, rows_per_w)], idx_v)

        def fire(g, b):
            for k in range(K):
                pltpu.async_copy(
                    table_hbm.at[idx_v.at[g * K + k]],
                    rows_v.at[b, pl.ds(k * IDXW, IDXW)],
                    gsem.at[b],
                )

        def drain_gathers(b):
            for k in range(K):
                pltpu.make_async_copy(
                    table_hbm.at[idx_v.at[0]],
                    rows_v.at[b, pl.ds(k * IDXW, IDXW)],
                    gsem.at[b],
                ).wait()

        def out_copy(g, b):
            return pltpu.async_copy(
                rows_v.at[b],
                out_hbm.at[pl.ds((row_base + g * K) * IDXW, CHUNK)],
                osem.at[b],
            )

        def drain_out(g, b):
            pltpu.make_async_copy(
                rows_v.at[b],
                out_hbm.at[pl.ds((row_base + g * K) * IDXW, CHUNK)],
                osem.at[b],
            ).wait()

        fire(0, 0)

        def body(gg, _):
            for b in range(NBUF):
                g = gg * NBUF + b
                nb = (b + 1) % NBUF

                @pl.when(g + 1 < n_chunks)
                def _():
                    @pl.when(g + 1 >= NBUF)
                    def _():
                        drain_out(g + 1 - NBUF, nb)

                    fire(g + 1, nb)

                drain_gathers(b)
                out_copy(g, b)
            return 0

        lax.fori_loop(0, n_chunks // NBUF, body, 0)

        for t in range(NBUF):
            g = n_chunks - NBUF + t
            drain_out(g, g % NBUF)

    return gather_kernel


def kernel(input_words, in_table):
    b, l = input_words.shape
    n = b * l
    idx2d = input_words.reshape(n // IDXW, IDXW).astype(jnp.int32)
    out = _make_gather(n)(idx2d, in_table)
    return out.reshape(b, l, DIM)


# speedup vs baseline: 1.6350x; 1.4692x over previous
import functools

import jax
import jax.numpy as jnp
from jax import lax
from jax.experimental import pallas as pl
from jax.experimental.pallas import tpu as pltpu
from jax.experimental.pallas import tpu_sc as plsc

DIM = 32
NC = 2
NS = 16
NW = NC * NS


def _make_gather(b, l):
    bpw = b // NW
    rows_pw = (bpw * l) // 128
    nbk = bpw // 128

    mesh = plsc.VectorSubcoreMesh(core_axis_name="c", subcore_axis_name="s")

    @functools.partial(
        pl.kernel,
        mesh=mesh,
        out_type=jax.ShapeDtypeStruct((l, DIM // 8, b // 128, 8, 128), jnp.float32),
        scratch_types=[
            pltpu.VMEM((rows_pw, 128), jnp.int32),
            pltpu.VMEM((l, bpw), jnp.int32),
            pltpu.VMEM((2, bpw, DIM), jnp.float32),
            pltpu.VMEM((2, DIM // 8, nbk, 8, 128), jnp.float32),
            pltpu.SemaphoreType.DMA((2,)),
            pltpu.SemaphoreType.DMA((2,)),
        ],
        compiler_params=pltpu.CompilerParams(
            use_tc_tiling_on_sc=False, needs_layout_passes=False
        ),
    )
    def gather_kernel(idx_hbm, table_hbm, out_hbm, idxv, lidx, rows, tbuf,
                      gsem, osem):
        wid = lax.axis_index("s") * NC + lax.axis_index("c")
        iota = lax.iota(jnp.int32, 16)

        pltpu.sync_copy(idx_hbm.at[pl.ds(wid * rows_pw, rows_pw)], idxv)

        v_l = iota * l

        def repack(li, _):
            def inner(c, _):
                p = v_l + (c * 16 * l + li)
                r = lax.shift_right_logical(p, 7)
                cc = lax.bitwise_and(p, 127)
                lidx[li, pl.ds(c * 16, 16)] = plsc.load_gather(idxv, [r, cc])
                return 0

            lax.fori_loop(0, bpw // 16, inner, 0)
            return 0

        lax.fori_loop(0, l, repack, 0)

        def fire_gathers(li, p):
            for k in range(nbk):
                pltpu.async_copy(
                    table_hbm.at[lidx.at[li, pl.ds(k * 128, 128)]],
                    rows.at[p, pl.ds(k * 128, 128)],
                    gsem.at[p],
                )

        def drain_gathers(p):
            for k in range(nbk):
                pltpu.make_async_copy(
                    table_hbm.at[lidx.at[0, pl.ds(0, 128)]],
                    rows.at[p, pl.ds(k * 128, 128)],
                    gsem.at[p],
                ).wait()

        def fire_writes(li, p):
            for g in range(DIM // 8):
                pltpu.async_copy(
                    tbuf.at[p, g],
                    out_hbm.at[li, g, pl.ds(nbk * wid, nbk)],
                    osem.at[p],
                )

        def drain_writes(li, p):
            for g in range(DIM // 8):
                pltpu.make_async_copy(
                    tbuf.at[p, g],
                    out_hbm.at[li, g, pl.ds(nbk * wid, nbk)],
                    osem.at[p],
                ).wait()

        def transpose(p):
            def inner(c, _):
                bv = c * 16 + iota
                cb = lax.shift_right_logical(bv, 7)
                cc = lax.bitwise_and(bv, 127)
                for d in range(DIM):
                    v = plsc.load_gather(rows.at[p], [bv, iota * 0 + d])
                    plsc.store_scatter(
                        tbuf.at[p, d // 8],
                        [cb, iota * 0 + (d % 8), cc],
                        v,
                    )
                return 0

            lax.fori_loop(0, bpw // 16, inner, 0)

        fire_gathers(0, 0)

        def body(ll, _):
            for sub in range(2):
                li = ll * 2 + sub
                p = sub

                @pl.when(li + 1 < l)
                def _():
                    fire_gathers(li + 1, 1 - p)

                drain_gathers(p)

                @pl.when(ll >= 1)
                def _():
                    drain_writes(li - 2, p)

                transpose(p)
                fire_writes(li, p)
            return 0

        lax.fori_loop(0, l // 2, body, 0)

        drain_writes(l - 2, 0)
        drain_writes(l - 1, 1)

    return gather_kernel


def kernel(input_words, in_table):
    b, l = input_words.shape
    n = b * l
    idx2d = input_words.reshape(n // 128, 128).astype(jnp.int32)
    y5 = _make_gather(b, l)(idx2d, in_table)
    t = y5.transpose(2, 4, 0, 1, 3)
    return t.reshape(b, l, DIM)


# speedup vs baseline: 1.6350x; 1.0000x over previous
import functools

import jax
import jax.numpy as jnp
from jax import lax
from jax.experimental import pallas as pl
from jax.experimental.pallas import tpu as pltpu
from jax.experimental.pallas import tpu_sc as plsc

DIM = 32
NC = 2
NS = 16
NW = NC * NS


def _make_gather(b, l):
    bpw = b // NW
    rows_pw = (bpw * l) // 128
    nbk = bpw // 128

    mesh = plsc.VectorSubcoreMesh(core_axis_name="c", subcore_axis_name="s")

    @functools.partial(
        pl.kernel,
        mesh=mesh,
        out_type=jax.ShapeDtypeStruct((l, DIM // 8, b // 128, 8, 128), jnp.float32),
        scratch_types=[
            pltpu.VMEM((rows_pw, 128), jnp.int32),
            pltpu.VMEM((l, bpw), jnp.int32),
            pltpu.VMEM((2, bpw, DIM), jnp.float32),
            pltpu.VMEM((2, DIM // 8, nbk, 8, 128), jnp.float32),
            pltpu.SemaphoreType.DMA((2,)),
            pltpu.SemaphoreType.DMA((2,)),
        ],
        compiler_params=pltpu.CompilerParams(
            use_tc_tiling_on_sc=False, needs_layout_passes=False
        ),
    )
    def gather_kernel(idx_hbm, table_hbm, out_hbm, idxv, lidx, rows, tbuf,
                      gsem, osem):
        wid = lax.axis_index("s") * NC + lax.axis_index("c")
        iota = lax.iota(jnp.int32, 16)

        pltpu.sync_copy(idx_hbm.at[pl.ds(wid * rows_pw, rows_pw)], idxv)

        v_l = iota * l

        def repack(li, _):
            def inner(c, _):
                p = v_l + (c * 16 * l + li)
                r = lax.shift_right_logical(p, 7)
                cc = lax.bitwise_and(p, 127)
                lidx[li, pl.ds(c * 16, 16)] = plsc.load_gather(idxv, [r, cc])
                return 0

            lax.fori_loop(0, bpw // 16, inner, 0)
            return 0

        lax.fori_loop(0, l, repack, 0)

        def fire_gathers(li, p):
            pltpu.async_copy(
                table_hbm.at[lidx.at[li]],
                rows.at[p],
                gsem.at[p],
            )

        def drain_gathers(p):
            pltpu.make_async_copy(
                table_hbm.at[lidx.at[0]],
                rows.at[p],
                gsem.at[p],
            ).wait()

        def fire_writes(li, p):
            for g in range(DIM // 8):
                pltpu.async_copy(
                    tbuf.at[p, g],
                    out_hbm.at[li, g, pl.ds(nbk * wid, nbk)],
                    osem.at[p],
                )

        def drain_writes(li, p):
            for g in range(DIM // 8):
                pltpu.make_async_copy(
                    tbuf.at[p, g],
                    out_hbm.at[li, g, pl.ds(nbk * wid, nbk)],
                    osem.at[p],
                ).wait()

        def transpose(p):
            def inner(c, _):
                bv = c * 16 + iota
                cb = lax.shift_right_logical(bv, 7)
                cc = lax.bitwise_and(bv, 127)
                for d in range(DIM):
                    v = plsc.load_gather(rows.at[p], [bv, iota * 0 + d])
                    plsc.store_scatter(
                        tbuf.at[p, d // 8],
                        [cb, iota * 0 + (d % 8), cc],
                        v,
                    )
                return 0

            lax.fori_loop(0, bpw // 16, inner, 0)

        fire_gathers(0, 0)

        def body(ll, _):
            for sub in range(2):
                li = ll * 2 + sub
                p = sub

                @pl.when(li + 1 < l)
                def _():
                    fire_gathers(li + 1, 1 - p)

                drain_gathers(p)

                @pl.when(ll >= 1)
                def _():
                    drain_writes(li - 2, p)

                transpose(p)
                fire_writes(li, p)
            return 0

        lax.fori_loop(0, l // 2, body, 0)

        drain_writes(l - 2, 0)
        drain_writes(l - 1, 1)

    return gather_kernel


def kernel(input_words, in_table):
    b, l = input_words.shape
    n = b * l
    idx2d = input_words.reshape(n // 128, 128).astype(jnp.int32)
    y5 = _make_gather(b, l)(idx2d, in_table)
    t = y5.transpose(2, 4, 0, 1, 3)
    return t.reshape(b, l, DIM)


# speedup vs baseline: 1.8216x; 1.1141x over previous
import functools

import jax
import jax.numpy as jnp
from jax import lax
from jax.experimental import pallas as pl
from jax.experimental.pallas import tpu as pltpu
from jax.experimental.pallas import tpu_sc as plsc

DIM = 32
NC = 2
NS = 16
NW = NC * NS


def _make_gather(b, l):
    bpw = b // NW
    rows_pw = (bpw * l) // 128
    nbk = bpw // 128

    mesh = plsc.VectorSubcoreMesh(core_axis_name="c", subcore_axis_name="s")

    @functools.partial(
        pl.kernel,
        mesh=mesh,
        out_type=jax.ShapeDtypeStruct((l, DIM // 8, b // 128, 8, 128), jnp.float32),
        scratch_types=[
            pltpu.VMEM((rows_pw, 128), jnp.int32),
            pltpu.VMEM((l, bpw), jnp.int32),
            pltpu.VMEM((2, bpw, DIM), jnp.float32),
            pltpu.VMEM((2, DIM // 8, nbk, 8, 128), jnp.float32),
            pltpu.SemaphoreType.DMA((2,)),
            pltpu.SemaphoreType.DMA((2,)),
        ],
        compiler_params=pltpu.CompilerParams(
            use_tc_tiling_on_sc=False, needs_layout_passes=False
        ),
    )
    def gather_kernel(idx_hbm, table_hbm, out_hbm, idxv, lidx, rows, tbuf,
                      gsem, osem):
        wid = lax.axis_index("s") * NC + lax.axis_index("c")
        iota = lax.iota(jnp.int32, 16)

        pltpu.sync_copy(idx_hbm.at[pl.ds(wid * rows_pw, rows_pw)], idxv)

        v_l = iota * l

        def repack(li, _):
            def inner(c, _):
                p = v_l + (c * 16 * l + li)
                r = lax.shift_right_logical(p, 7)
                cc = lax.bitwise_and(p, 127)
                lidx[li, pl.ds(c * 16, 16)] = plsc.load_gather(idxv, [r, cc])
                return 0

            lax.fori_loop(0, bpw // 16, inner, 0)
            return 0

        lax.fori_loop(0, l, repack, 0)

        def fire_gathers(li, p):
            pltpu.async_copy(
                table_hbm.at[lidx.at[li]],
                rows.at[p],
                gsem.at[p],
            )

        def drain_gathers(p):
            pltpu.make_async_copy(
                table_hbm.at[lidx.at[0]],
                rows.at[p],
                gsem.at[p],
            ).wait()

        def fire_writes(li, p):
            for g in range(DIM // 8):
                pltpu.async_copy(
                    tbuf.at[p, g],
                    out_hbm.at[li, g, pl.ds(nbk * wid, nbk)],
                    osem.at[p],
                )

        def drain_writes(li, p):
            for g in range(DIM // 8):
                pltpu.make_async_copy(
                    tbuf.at[p, g],
                    out_hbm.at[li, g, pl.ds(nbk * wid, nbk)],
                    osem.at[p],
                ).wait()

        g_vec = [lax.shift_right_logical(h * 16 + iota, 3) for h in range(2)]
        s_vec = [lax.bitwise_and(h * 16 + iota, 7) for h in range(2)]

        def transpose(p):
            def inner(rr, _):
                for i in range(16):
                    bq = rr * 16 + i
                    cb = iota * 0 + lax.shift_right_logical(bq, 7)
                    cc = iota * 0 + lax.bitwise_and(bq, 127)
                    for h in range(2):
                        v = rows[p, bq, pl.ds(h * 16, 16)]
                        plsc.store_scatter(
                            tbuf.at[p], [g_vec[h], cb, s_vec[h], cc], v
                        )
                return 0

            lax.fori_loop(0, bpw // 16, inner, 0)

        fire_gathers(0, 0)

        def body(ll, _):
            for sub in range(2):
                li = ll * 2 + sub
                p = sub

                @pl.when(li + 1 < l)
                def _():
                    fire_gathers(li + 1, 1 - p)

                drain_gathers(p)

                @pl.when(ll >= 1)
                def _():
                    drain_writes(li - 2, p)

                transpose(p)
                fire_writes(li, p)
            return 0

        lax.fori_loop(0, l // 2, body, 0)

        drain_writes(l - 2, 0)
        drain_writes(l - 1, 1)

    return gather_kernel


def kernel(input_words, in_table):
    b, l = input_words.shape
    n = b * l
    idx2d = input_words.reshape(n // 128, 128).astype(jnp.int32)
    y5 = _make_gather(b, l)(idx2d, in_table)
    t = y5.transpose(2, 4, 0, 1, 3)
    return t.reshape(b, l, DIM)


# speedup vs baseline: 1.8243x; 1.0015x over previous
import functools

import jax
import jax.numpy as jnp
from jax import lax
from jax.experimental import pallas as pl
from jax.experimental.pallas import tpu as pltpu
from jax.experimental.pallas import tpu_sc as plsc

DIM = 32
NC = 2
NS = 16
NW = NC * NS


def _make_gather(b, l):
    bpw = b // NW
    rows_pw = (bpw * l) // 128
    nbk = bpw // 128

    mesh = plsc.VectorSubcoreMesh(core_axis_name="c", subcore_axis_name="s")

    @functools.partial(
        pl.kernel,
        mesh=mesh,
        out_type=jax.ShapeDtypeStruct((l * DIM * b,), jnp.float32),
        scratch_types=[
            pltpu.VMEM((rows_pw, 128), jnp.int32),
            pltpu.VMEM((l, bpw), jnp.int32),
            pltpu.VMEM((2, bpw, DIM), jnp.float32),
            pltpu.VMEM((2, 2, 2 * nbk * 1024), jnp.float32),
            pltpu.SemaphoreType.DMA((2,)),
            pltpu.SemaphoreType.DMA((2,)),
        ],
        compiler_params=pltpu.CompilerParams(
            use_tc_tiling_on_sc=False, needs_layout_passes=False
        ),
    )
    def gather_kernel(idx_hbm, table_hbm, out_hbm, idxv, lidx, rows, tbuf,
                      gsem, osem):
        wid = lax.axis_index("s") * NC + lax.axis_index("c")
        iota = lax.iota(jnp.int32, 16)

        pltpu.sync_copy(idx_hbm.at[pl.ds(wid * rows_pw, rows_pw)], idxv)

        v_l = iota * l

        def repack(li, _):
            def inner(c, _):
                p = v_l + (c * 16 * l + li)
                r = lax.shift_right_logical(p, 7)
                cc = lax.bitwise_and(p, 127)
                lidx[li, pl.ds(c * 16, 16)] = plsc.load_gather(idxv, [r, cc])
                return 0

            lax.fori_loop(0, bpw // 16, inner, 0)
            return 0

        lax.fori_loop(0, l, repack, 0)

        def fire_gathers(li, p):
            pltpu.async_copy(
                table_hbm.at[lidx.at[li]],
                rows.at[p],
                gsem.at[p],
            )

        def drain_gathers(p):
            pltpu.make_async_copy(
                table_hbm.at[lidx.at[0]],
                rows.at[p],
                gsem.at[p],
            ).wait()

        gsz = nbk * 1024

        def fire_writes(li, p):
            for g in range(DIM // 8):
                off = li * b * DIM + g * b * 8 + wid * gsz
                pltpu.async_copy(
                    tbuf.at[p, g // 2, pl.ds((g % 2) * gsz, gsz)],
                    out_hbm.at[pl.ds(off, gsz)],
                    osem.at[p],
                )

        def drain_writes(li, p):
            for g in range(DIM // 8):
                off = li * b * DIM + g * b * 8 + wid * gsz
                pltpu.make_async_copy(
                    tbuf.at[p, g // 2, pl.ds((g % 2) * gsz, gsz)],
                    out_hbm.at[pl.ds(off, gsz)],
                    osem.at[p],
                ).wait()

        pb = lax.shift_right_logical(iota, 3) * 4096 + lax.bitwise_and(iota, 7) * 128

        def transpose(p):
            def inner(rr, _):
                for i in range(16):
                    bq = rr * 16 + i
                    base = lax.shift_right_logical(bq, 7) * 1024 + lax.bitwise_and(bq, 127)
                    pos = pb + base
                    for h in range(2):
                        v = rows[p, bq, pl.ds(h * 16, 16)]
                        plsc.store_scatter(tbuf.at[p, h], [pos], v)
                return 0

            lax.fori_loop(0, bpw // 16, inner, 0)

        fire_gathers(0, 0)

        def body(ll, _):
            for sub in range(2):
                li = ll * 2 + sub
                p = sub

                @pl.when(li + 1 < l)
                def _():
                    fire_gathers(li + 1, 1 - p)

                drain_gathers(p)

                @pl.when(ll >= 1)
                def _():
                    drain_writes(li - 2, p)

                transpose(p)
                fire_writes(li, p)
            return 0

        lax.fori_loop(0, l // 2, body, 0)

        drain_writes(l - 2, 0)
        drain_writes(l - 1, 1)

    return gather_kernel


def kernel(input_words, in_table):
    b, l = input_words.shape
    n = b * l
    idx2d = input_words.reshape(n // 128, 128).astype(jnp.int32)
    y = _make_gather(b, l)(idx2d, in_table)
    y5 = y.reshape(l, DIM // 8, b // 128, 8, 128)
    t = y5.transpose(2, 4, 0, 1, 3)
    return t.reshape(b, l, DIM)


# speedup vs baseline: 2.1302x; 1.1676x over previous
import functools

import jax
import jax.numpy as jnp
from jax import lax
from jax.experimental import pallas as pl
from jax.experimental.pallas import tpu as pltpu
from jax.experimental.pallas import tpu_sc as plsc

DIM = 32
NC = 2
NS = 16
NW = NC * NS


def _make_gather(b, l):
    bpw = b // NW
    rows_pw = (bpw * l) // 128
    nbk = bpw // 128

    mesh = plsc.VectorSubcoreMesh(core_axis_name="c", subcore_axis_name="s")

    @functools.partial(
        pl.kernel,
        mesh=mesh,
        out_type=jax.ShapeDtypeStruct((l * DIM * b,), jnp.float32),
        scratch_types=[
            pltpu.VMEM((rows_pw, 128), jnp.int32),
            pltpu.VMEM((l, bpw), jnp.int32),
            pltpu.VMEM((2, bpw, DIM), jnp.float32),
            pltpu.VMEM((2, 2, 2 * nbk * 1024), jnp.float32),
            pltpu.SemaphoreType.DMA((2,)),
            pltpu.SemaphoreType.DMA((2,)),
        ],
        compiler_params=pltpu.CompilerParams(
            use_tc_tiling_on_sc=False, needs_layout_passes=False
        ),
    )
    def gather_kernel(idx_hbm, table_hbm, out_hbm, idxv, lidx, rows, tbuf,
                      gsem, osem):
        wid = lax.axis_index("s") * NC + lax.axis_index("c")
        iota = lax.iota(jnp.int32, 16)

        pltpu.sync_copy(idx_hbm.at[pl.ds(wid * rows_pw, rows_pw)], idxv)

        v_l = iota * l

        def repack(li, _):
            def inner(c, _):
                p = v_l + (c * 16 * l + li)
                r = lax.shift_right_logical(p, 7)
                cc = lax.bitwise_and(p, 127)
                lidx[li, pl.ds(c * 16, 16)] = plsc.load_gather(idxv, [r, cc])
                return 0

            lax.fori_loop(0, bpw // 16, inner, 0)
            return 0

        lax.fori_loop(0, l, repack, 0)

        def fire_gathers(li, p):
            pltpu.async_copy(
                table_hbm.at[lidx.at[li]],
                rows.at[p],
                gsem.at[p],
            )

        def drain_gathers(p):
            pltpu.make_async_copy(
                table_hbm.at[lidx.at[0]],
                rows.at[p],
                gsem.at[p],
            ).wait()

        gsz = nbk * 1024

        def fire_writes(li, p):
            for g in range(DIM // 8):
                off = li * b * DIM + g * b * 8 + wid * gsz
                pltpu.async_copy(
                    tbuf.at[p, g // 2, pl.ds((g % 2) * gsz, gsz)],
                    out_hbm.at[pl.ds(off, gsz)],
                    osem.at[p],
                )

        def drain_writes(li, p):
            for g in range(DIM // 8):
                off = li * b * DIM + g * b * 8 + wid * gsz
                pltpu.make_async_copy(
                    tbuf.at[p, g // 2, pl.ds((g % 2) * gsz, gsz)],
                    out_hbm.at[pl.ds(off, gsz)],
                    osem.at[p],
                ).wait()

        pb = lax.shift_right_logical(iota, 3) * 4096 + lax.bitwise_and(iota, 7) * 128

        def transpose(p):
            @plsc.parallel_loop(0, bpw, unroll=8)
            def _(bq):
                base = lax.shift_right_logical(bq, 7) * 1024 + lax.bitwise_and(bq, 127)
                pos = pb + base
                for h in range(2):
                    v = rows[p, bq, pl.ds(h * 16, 16)]
                    plsc.store_scatter(tbuf.at[p, h], [pos], v)

        fire_gathers(0, 0)

        def body(ll, _):
            for sub in range(2):
                li = ll * 2 + sub
                p = sub

                @pl.when(li + 1 < l)
                def _():
                    fire_gathers(li + 1, 1 - p)

                drain_gathers(p)

                @pl.when(ll >= 1)
                def _():
                    drain_writes(li - 2, p)

                transpose(p)
                fire_writes(li, p)
            return 0

        lax.fori_loop(0, l // 2, body, 0)

        drain_writes(l - 2, 0)
        drain_writes(l - 1, 1)

    return gather_kernel


def kernel(input_words, in_table):
    b, l = input_words.shape
    n = b * l
    idx2d = input_words.reshape(n // 128, 128).astype(jnp.int32)
    y = _make_gather(b, l)(idx2d, in_table)
    y5 = y.reshape(l, DIM // 8, b // 128, 8, 128)
    t = y5.transpose(2, 4, 0, 1, 3)
    return t.reshape(b, l, DIM)


# speedup vs baseline: 2.1348x; 1.0022x over previous
import functools

import jax
import jax.numpy as jnp
from jax import lax
from jax.experimental import pallas as pl
from jax.experimental.pallas import tpu as pltpu
from jax.experimental.pallas import tpu_sc as plsc

DIM = 32
NC = 2
NS = 16
NW = NC * NS


def _make_gather(b, l):
    bpw = b // NW
    rows_pw = (bpw * l) // 128
    nbk = bpw // 128

    mesh = plsc.VectorSubcoreMesh(core_axis_name="c", subcore_axis_name="s")

    @functools.partial(
        pl.kernel,
        mesh=mesh,
        out_type=jax.ShapeDtypeStruct((l * DIM * b,), jnp.float32),
        scratch_types=[
            pltpu.VMEM((rows_pw, 128), jnp.int32),
            pltpu.VMEM((l, bpw), jnp.int32),
            pltpu.VMEM((2, bpw, DIM), jnp.float32),
            pltpu.VMEM((2, 2, 2 * nbk * 1024), jnp.float32),
            pltpu.SemaphoreType.DMA((2,)),
            pltpu.SemaphoreType.DMA((2,)),
        ],
        compiler_params=pltpu.CompilerParams(
            use_tc_tiling_on_sc=False, needs_layout_passes=False
        ),
    )
    def gather_kernel(idx_hbm, table_hbm, out_hbm, idxv, lidx, rows, tbuf,
                      gsem, osem):
        wid = lax.axis_index("s") * NC + lax.axis_index("c")
        iota = lax.iota(jnp.int32, 16)

        pltpu.sync_copy(idx_hbm.at[pl.ds(wid * rows_pw, rows_pw)], idxv)

        v_l = iota * l

        def repack(li, _):
            def inner(c, _):
                p = v_l + (c * 16 * l + li)
                r = lax.shift_right_logical(p, 7)
                cc = lax.bitwise_and(p, 127)
                lidx[li, pl.ds(c * 16, 16)] = plsc.load_gather(idxv, [r, cc])
                return 0

            lax.fori_loop(0, bpw // 16, inner, 0)
            return 0

        lax.fori_loop(0, l, repack, 0)

        def fire_gathers(li, p):
            pltpu.async_copy(
                table_hbm.at[lidx.at[li]],
                rows.at[p],
                gsem.at[p],
            )

        def drain_gathers(p):
            pltpu.make_async_copy(
                table_hbm.at[lidx.at[0]],
                rows.at[p],
                gsem.at[p],
            ).wait()

        gsz = nbk * 1024

        def fire_writes(li, p):
            for g in range(DIM // 8):
                off = li * b * DIM + g * b * 8 + wid * gsz
                pltpu.async_copy(
                    tbuf.at[p, g // 2, pl.ds((g % 2) * gsz, gsz)],
                    out_hbm.at[pl.ds(off, gsz)],
                    osem.at[p],
                )

        def drain_writes(li, p):
            for g in range(DIM // 8):
                off = li * b * DIM + g * b * 8 + wid * gsz
                pltpu.make_async_copy(
                    tbuf.at[p, g // 2, pl.ds((g % 2) * gsz, gsz)],
                    out_hbm.at[pl.ds(off, gsz)],
                    osem.at[p],
                ).wait()

        pb = lax.shift_right_logical(iota, 3) * 4096 + lax.bitwise_and(iota, 7) * 128

        def transpose(p):
            @plsc.parallel_loop(0, bpw, unroll=16)
            def _(bq):
                base = lax.shift_right_logical(bq, 7) * 1024 + lax.bitwise_and(bq, 127)
                pos = pb + base
                for h in range(2):
                    v = rows[p, bq, pl.ds(h * 16, 16)]
                    plsc.store_scatter(tbuf.at[p, h], [pos], v)

        fire_gathers(0, 0)

        def body(ll, _):
            for sub in range(2):
                li = ll * 2 + sub
                p = sub

                @pl.when(li + 1 < l)
                def _():
                    fire_gathers(li + 1, 1 - p)

                drain_gathers(p)

                @pl.when(ll >= 1)
                def _():
                    drain_writes(li - 2, p)

                transpose(p)
                fire_writes(li, p)
            return 0

        lax.fori_loop(0, l // 2, body, 0)

        drain_writes(l - 2, 0)
        drain_writes(l - 1, 1)

    return gather_kernel


def kernel(input_words, in_table):
    b, l = input_words.shape
    n = b * l
    idx2d = input_words.reshape(n // 128, 128).astype(jnp.int32)
    y = _make_gather(b, l)(idx2d, in_table)
    y5 = y.reshape(l, DIM // 8, b // 128, 8, 128)
    t = y5.transpose(2, 4, 0, 1, 3)
    return t.reshape(b, l, DIM)
